# baseline (speedup 1.0000x reference)
import functools

import jax
import jax.numpy as jnp
from jax import lax
from jax.experimental import pallas as pl
from jax.experimental.pallas import tpu as pltpu
from jax.experimental.pallas import tpu_sc as plsc

_SCALE = 0.02
_K = 10
_BLK = 128
_L = 128
_DEPTH = 5

_INTERPRET = False


def _pick_loop(c_vals, c_cols, pos, blk, sentinel):
    inf = jnp.float32(jnp.inf)
    lane16 = lax.broadcasted_iota(jnp.int32, (blk, 16), 1)

    def step(k, carry):
        g, colp, vals_acc, idx_acc, col_acc = carry
        dup_ok = (c_vals == g) & (c_cols > colp)
        stay_a = jnp.min(jnp.where(dup_ok, c_cols, sentinel), axis=1,
                         keepdims=True)
        adv_b = jnp.min(jnp.where(c_vals > g, c_vals, inf), axis=1,
                        keepdims=True)
        col_b = jnp.min(jnp.where(c_vals == adv_b, c_cols, sentinel), axis=1,
                        keepdims=True)
        stay = stay_a < sentinel
        g_k = jnp.where(stay, g, adv_b)
        col_k = jnp.where(stay, stay_a, col_b)
        kept = col_k - (col_k > pos).astype(jnp.int32)
        vals_acc = jnp.where(lane16 == k, g_k, vals_acc)
        idx_acc = jnp.where(lane16 == k, kept, idx_acc)
        col_acc = jnp.where(lane16 == k, col_k, col_acc)
        return g_k, col_k, vals_acc, idx_acc, col_acc

    init = (jnp.full((blk, 1), -inf, jnp.float32),
            jnp.full((blk, 1), -1, jnp.int32),
            jnp.zeros((blk, 16), jnp.float32),
            jnp.zeros((blk, 16), jnp.int32),
            jnp.zeros((blk, 16), jnp.int32))
    _, _, vals_acc, idx_acc, col_acc = lax.fori_loop(0, _K, step, init,
                                                     unroll=False)
    return vals_acc, idx_acc, col_acc


def _dist_topk_body(n_total, blk, f_ref, c_ref, pos_ref, vals_ref, idx_ref,
                    csq_ref):
    n_seg = n_total // _L
    i = pl.program_id(1)

    @pl.when(i == 0)
    def _():
        c = c_ref[0]
        csq_ref[0:1, :] = jnp.sum(c * c, axis=1)[None, :]

    f = f_ref[0]
    pos = pos_ref[:, 0:1]

    fsq = jnp.sum(f * f, axis=1, keepdims=True)
    fc = lax.dot_general(f, c_ref[0], dimension_numbers=(((1,), (1,)), ((), ())),
                         preferred_element_type=jnp.float32)
    cols = lax.broadcasted_iota(jnp.int32, (blk, n_total), 1)
    inf = jnp.float32(jnp.inf)
    dist = fsq + csq_ref[0:1, :] - 2.0 * fc
    dist = jnp.where(cols == pos, inf, dist)

    y_sum = jnp.sum(jnp.exp(dist * (-_SCALE)), axis=1, keepdims=True)

    lane16 = lax.broadcasted_iota(jnp.int32, (blk, 16), 1)

    def finalize(vals_acc, idx_acc):
        x_sum = jnp.sum(
            jnp.where(lane16 < _K, jnp.exp(vals_acc * (-_SCALE)), 0.0),
            axis=1, keepdims=True)
        rt = -jnp.log(x_sum) + jnp.log(y_sum)
        vals_ref[0] = jnp.where(lane16 == _K, rt, vals_acc)
        idx_ref[0] = idx_acc

    d3 = dist.reshape(blk, n_seg, _L)
    lane3 = lax.broadcasted_iota(jnp.int32, (blk, n_seg, _L), 2)
    seg_base = lax.broadcasted_iota(jnp.int32, (blk, n_seg), 1) * _L
    d5 = lax.broadcasted_iota(jnp.int32, (blk, _DEPTH, n_seg), 1)

    def level(t, carry):
        work, acc_v, acc_c = carry
        m = jnp.min(work, axis=2)
        a = jnp.min(jnp.where(work == m[..., None], lane3, _L), axis=2)
        acc_v = jnp.where(d5 == t, m[:, None, :], acc_v)
        acc_c = jnp.where(d5 == t, (seg_base + a)[:, None, :], acc_c)
        work = jnp.where(lane3 == a[..., None], inf, work)
        return work, acc_v, acc_c

    linit = (d3, jnp.zeros((blk, _DEPTH, n_seg), jnp.float32),
             jnp.zeros((blk, _DEPTH, n_seg), jnp.int32))
    _, acc_v, acc_c = lax.fori_loop(0, _DEPTH, level, linit, unroll=False)
    cand_v = acc_v.reshape(blk, _DEPTH * n_seg)
    cand_c = acc_c.reshape(blk, _DEPTH * n_seg)
    vals_acc, idx_acc, col_acc = _pick_loop(cand_v, cand_c, pos, blk, n_total)
    finalize(vals_acc, idx_acc)

    segs = col_acc // _L
    viol_row = jnp.zeros((blk, 1), jnp.bool_)
    for k in range(_K):
        seg_k = jnp.sum(jnp.where(lane16 == k, segs, 0), axis=1, keepdims=True)
        cnt_k = jnp.sum(jnp.where((lane16 < _K) & (segs == seg_k), 1, 0),
                        axis=1, keepdims=True)
        viol_row = viol_row | (cnt_k >= _DEPTH)
    viol = jnp.any(viol_row)

    @pl.when(viol)
    def _():
        vals2, idx2, _ = _pick_loop(dist, cols, pos, blk, n_total)
        finalize(vals2, idx2)


def _tc_dist_topk(feature, centers, position):
    p_dim, b_dim, d_dim = feature.shape
    n_dim = centers.shape[1]
    blk = _BLK
    nb = b_dim // blk
    grid = (p_dim, nb)
    body = functools.partial(_dist_topk_body, n_dim, blk)
    vals, idx = pl.pallas_call(
        body,
        grid=grid,
        in_specs=[
            pl.BlockSpec((1, blk, d_dim), lambda p, i: (p, i, 0)),
            pl.BlockSpec((1, n_dim, d_dim), lambda p, i: (p, 0, 0)),
            pl.BlockSpec((blk, 1), lambda p, i: (i, 0)),
        ],
        out_specs=[
            pl.BlockSpec((1, blk, 16), lambda p, i: (p, i, 0)),
            pl.BlockSpec((1, blk, 16), lambda p, i: (p, i, 0)),
        ],
        out_shape=[
            jax.ShapeDtypeStruct((p_dim, b_dim, 16), jnp.float32),
            jax.ShapeDtypeStruct((p_dim, b_dim, 16), jnp.int32),
        ],
        scratch_shapes=[pltpu.VMEM((8, n_dim), jnp.float32)],
        interpret=_INTERPRET,
    )(feature, centers, position.reshape(b_dim, 1))
    return vals, idx


def _sc_gather(pm_pid, idx_flat):
    n_dim = pm_pid.shape[0]
    tot = idx_flat.shape[0]
    info = plsc.get_sparse_core_info()
    nw = info.num_cores * info.num_subcores
    lanes = info.num_lanes
    chunk = tot // nw
    mesh = plsc.VectorSubcoreMesh(core_axis_name="c", subcore_axis_name="s")

    @functools.partial(
        pl.kernel,
        mesh=mesh,
        out_type=jax.ShapeDtypeStruct((tot,), jnp.int32),
        scratch_types=[
            pltpu.VMEM((n_dim,), jnp.int32),
            pltpu.VMEM((chunk,), jnp.int32),
            pltpu.VMEM((chunk,), jnp.int32),
        ],
        compiler_params=pltpu.CompilerParams(needs_layout_passes=False),
    )
    def gk(pid_hbm, idx_hbm, out_hbm, table_v, idx_v, outs_v):
        wid = lax.axis_index("s") * info.num_cores + lax.axis_index("c")
        base = wid * chunk
        pltpu.sync_copy(pid_hbm, table_v)
        pltpu.sync_copy(idx_hbm.at[pl.ds(base, chunk)], idx_v)

        def body(j, carry):
            iv = idx_v[pl.ds(j * lanes, lanes)]
            outs_v[pl.ds(j * lanes, lanes)] = plsc.load_gather(table_v, [iv])
            return carry

        lax.fori_loop(0, chunk // lanes, body, 0)
        pltpu.sync_copy(outs_v, out_hbm.at[pl.ds(base, chunk)])

    return gk(pm_pid, idx_flat)


def kernel(feature, centers, position, pm_camid, pm_pid, camid):
    p_dim, b_dim, _ = feature.shape
    vals, kidx = _tc_dist_topk(feature, centers, position)
    rt = vals[:, :, _K]
    l_p = jnp.sum(rt, axis=1) / b_dim
    l_p = jnp.where(jnp.isnan(l_p), jnp.zeros_like(l_p), l_p)
    loss = jnp.sum(l_p) / p_dim
    idx_flat = kidx[:, :, :_K].reshape(-1)
    pos_vid = _sc_gather(pm_pid, idx_flat).reshape(p_dim, b_dim, _K)
    return (loss, pos_vid)
